# baseline (speedup 1.0000x reference)
import functools

import jax
import jax.numpy as jnp
from jax import lax
from jax.experimental import pallas as pl
from jax.experimental.pallas import tpu as pltpu
from jax.experimental.pallas import tpu_sc as plsc

_LANES = 16
_NC = 2
_NS = 16
_CHUNK = 128
_NF_PAD = 10240
_ROWS_PER_TILE = 640
_COPY_ROWS = 128
_CROWS = _NF_PAD // 128


def _sc_segment_sum(x, edge_src, edge_dst):
  e = edge_src.shape[0]
  d = x.shape[1]
  nw = _NC * _NS
  per_tile = e // nw
  n_chunks = -(-per_tile // _CHUNK)

  mesh = plsc.VectorSubcoreMesh(core_axis_name="c", subcore_axis_name="s")

  @functools.partial(
      pl.kernel,
      mesh=mesh,
      compiler_params=pltpu.CompilerParams(needs_layout_passes=False),
      out_type=[
          jax.ShapeDtypeStruct((_NC, _NF_PAD, d), jnp.float32),
          jax.ShapeDtypeStruct((_NC, _CROWS, 128), jnp.float32),
      ],
      scratch_types=[
          pltpu.VMEM_SHARED((_NF_PAD, d), jnp.float32),
          pltpu.VMEM_SHARED((_CROWS, 128), jnp.float32),
          pltpu.VMEM((2, _CHUNK), jnp.int32),
          pltpu.VMEM((4, _CHUNK), jnp.int32),
          pltpu.VMEM((2, _CHUNK, d), jnp.float32),
          pltpu.VMEM((_CROWS + 1, 128), jnp.float32),
          pltpu.VMEM((_CROWS,), jnp.int32),
      ] + [pltpu.SemaphoreType.DMA] * 10,
  )
  def k(x_hbm, src_hbm, dst_hbm, out_hbm, cnt_hbm, acc_sh, cnt_sh, sidx_v,
        didx_v, rows2_v, hist_v, rowid_v, *sems):
    cid = lax.axis_index("c")
    sid = lax.axis_index("s")
    wid = cid * _NS + sid
    row0 = sid * _ROWS_PER_TILE
    isem = list(sems[0:2])
    dsem = list(sems[2:6])
    gsem = list(sems[6:8])
    ssem = list(sems[8:10])

    iota = lax.iota(jnp.int32, _LANES)
    zvec = jnp.zeros((_LANES,), jnp.float32)
    lane15 = iota == _LANES - 1
    shift_ix = [jnp.maximum(iota - k, 0)[:, None] for k in (1, 2, 4, 8)]
    nxt_ix = jnp.minimum(iota + 1, _LANES - 1)[:, None]
    ones16 = jnp.ones((_LANES,), jnp.int32)
    zeros16 = jnp.zeros((_LANES,), jnp.int32)
    gd = lax.GatherDimensionNumbers(
        offset_dims=(), collapsed_slice_dims=(0,), start_index_map=(0,))

    def shift(v, ix):
      return lax.gather(v, ix, gd, (1,),
                        mode=lax.GatherScatterMode.PROMISE_IN_BOUNDS)

    def zero_row(r, _):
      for j in range(d // _LANES):
        rows2_v[0, r, pl.ds(j * _LANES, _LANES)] = zvec
      return 0

    lax.fori_loop(0, _CHUNK, zero_row, 0)
    for blk in range(_ROWS_PER_TILE // _CHUNK):
      pltpu.sync_copy(rows2_v.at[0],
                      acc_sh.at[pl.ds(row0 + blk * _CHUNK, _CHUNK), :])

    def zero_hist(r, _):
      for j in range(128 // _LANES):
        hist_v[r, pl.ds(j * _LANES, _LANES)] = zvec
      return 0

    lax.fori_loop(0, _CROWS + 1, zero_hist, 0)
    for g in range(_CROWS // _LANES):
      rowid_v[pl.ds(g * _LANES, _LANES)] = iota + g * _LANES

    @pl.when(sid == 0)
    def _():
      pltpu.sync_copy(rows2_v.at[0, pl.ds(0, _CROWS), :], cnt_sh)

    plsc.subcore_barrier()

    def counts(slot):
      for g in range(_CHUNK // _LANES):
        dstv = didx_v[slot, pl.ds(g * _LANES, _LANES)]
        prv = shift(dstv, shift_ix[0])
        nxt = shift(dstv, nxt_ix)
        is_last = (dstv != nxt) | lane15
        m = jnp.where((dstv == prv) & (iota > 0), ones16, zeros16)
        c = ones16
        for ix in shift_ix:
          c = c + m * shift(c, ix)
          m = m * shift(m, ix)
        cnt = c.astype(jnp.float32)
        rows = jnp.where(is_last, lax.shift_right_logical(dstv, 7),
                         jnp.full((_LANES,), _CROWS, jnp.int32))
        plsc.addupdate_scatter(hist_v, [rows, lax.bitwise_and(dstv, 127)],
                               cnt)

    base = wid * n_chunks

    def fire_sidx(c, b):
      pltpu.async_copy(src_hbm.at[base + c], sidx_v.at[b], isem[b])

    def wait_sidx(c, b):
      pltpu.make_async_copy(src_hbm.at[base + c], sidx_v.at[b],
                            isem[b]).wait()

    def fire_didx(c, slot):
      pltpu.async_copy(dst_hbm.at[base + c], didx_v.at[slot], dsem[slot])

    def wait_didx(c, slot):
      pltpu.make_async_copy(dst_hbm.at[base + c], didx_v.at[slot],
                            dsem[slot]).wait()

    def fire_gather(b):
      pltpu.async_copy(x_hbm.at[sidx_v.at[b]], rows2_v.at[b], gsem[b])

    def wait_gather(b):
      pltpu.make_async_copy(x_hbm.at[sidx_v.at[b]], rows2_v.at[b],
                            gsem[b]).wait()

    def fire_scatter(slot, b):
      pltpu.async_copy(rows2_v.at[b], acc_sh.at[didx_v.at[slot]], ssem[b],
                       add=True)

    def wait_scatter(slot, b):
      pltpu.make_async_copy(rows2_v.at[b], acc_sh.at[didx_v.at[slot]],
                            ssem[b]).wait()

    def step(c, j, dyn):
      b = j % 2
      slot = j % 4
      wait_gather(b)

      def do_sidx():
        fire_sidx(c + 2, b)

      def do_didx():
        fire_didx(c + 4, slot)

      def do_gather():
        wait_sidx(c + 2, b)
        fire_gather(b)

      if dyn:
        pl.when(c + 2 < n_chunks)(do_sidx)
      elif c + 2 < n_chunks:
        do_sidx()
      wait_didx(c, slot)
      fire_scatter(slot, b)
      counts(slot)
      wait_scatter(slot, b)
      if dyn:
        pl.when(c + 4 < n_chunks)(do_didx)
      elif c + 4 < n_chunks:
        do_didx()
      if dyn:
        pl.when(c + 2 < n_chunks)(do_gather)
      elif c + 2 < n_chunks:
        do_gather()

    for j in range(2):
      fire_sidx(j, j)
    for j in range(4):
      fire_didx(j, j)
    for j in range(2):
      wait_sidx(j, j)
      fire_gather(j)

    def quad_body(k4, _):
      c0 = k4 * 4
      for j in range(4):
        step(c0 + j, j, True)
      return 0

    n_loop = n_chunks // 4
    lax.fori_loop(0, n_loop, quad_body, 0)
    for c in range(n_loop * 4, n_chunks):
      step(c, c % 4, False)

    pltpu.sync_copy(hist_v.at[pl.ds(0, _CROWS), :], cnt_sh.at[rowid_v],
                    add=True)
    plsc.subcore_barrier()

    pltpu.sync_copy(acc_sh.at[pl.ds(row0, _ROWS_PER_TILE), :],
                    out_hbm.at[cid, pl.ds(row0, _ROWS_PER_TILE), :])

    @pl.when(sid == 0)
    def _():
      pltpu.sync_copy(cnt_sh, cnt_hbm.at[cid])

  pad = n_chunks * _CHUNK - per_tile
  src2 = jnp.pad(edge_src.reshape(nw, per_tile),
                 ((0, 0), (0, pad))).reshape(nw * n_chunks, _CHUNK)
  dst2 = jnp.pad(edge_dst.reshape(nw, per_tile), ((0, 0), (0, pad)),
                 constant_values=_NF_PAD - 1).reshape(nw * n_chunks, _CHUNK)
  return k(x, src2, dst2)


def _tc_head(partials, counts, x_skip, w1, w2, b2):
  n_fine, d = x_skip.shape
  bm = 2000
  grid = (n_fine // bm,)

  def body(s_ref, c_ref, xs_ref, w1_ref, w2_ref, b_ref, o_ref):
    s = s_ref[0] + s_ref[1]
    cnt = c_ref[0] + c_ref[1]
    agg = s * (1.0 / jnp.maximum(cnt, 1.0))
    acc = jnp.dot(agg, w1_ref[...], preferred_element_type=jnp.float32)
    acc = acc + jnp.dot(xs_ref[...], w2_ref[...],
                        preferred_element_type=jnp.float32)
    o_ref[...] = jnp.maximum(acc + b_ref[...], 0.0)

  return pl.pallas_call(
      body,
      grid=grid,
      in_specs=[
          pl.BlockSpec((_NC, bm, d), lambda i: (0, i, 0)),
          pl.BlockSpec((_NC, bm, 1), lambda i: (0, i, 0)),
          pl.BlockSpec((bm, d), lambda i: (i, 0)),
          pl.BlockSpec((d, d), lambda i: (0, 0)),
          pl.BlockSpec((d, d), lambda i: (0, 0)),
          pl.BlockSpec((1, d), lambda i: (0, 0)),
      ],
      out_specs=pl.BlockSpec((bm, d), lambda i: (i, 0)),
      out_shape=jax.ShapeDtypeStruct((n_fine, d), jnp.float32),
  )(partials, counts, x_skip, w1, w2, b2)


def kernel(x, pos, x_skip, pos_skip, W, b, batch, batch_skip, edge_src,
           edge_dst):
  n_fine, d = x_skip.shape
  partials, cnt_packed = _sc_segment_sum(x, edge_src, edge_dst)
  counts = cnt_packed.reshape(_NC, _NF_PAD, 1)
  return _tc_head(partials, counts, x_skip, W[:d], W[d:], b.reshape(1, -1))


# speedup vs baseline: 1.9239x; 1.9239x over previous
import functools

import jax
import jax.numpy as jnp
from jax import lax
from jax.experimental import pallas as pl
from jax.experimental.pallas import tpu as pltpu
from jax.experimental.pallas import tpu_sc as plsc

_LANES = 16
_NC = 2
_NS = 16
_CHUNK = 80
_NF_PAD = 10240
_ROWS_PER_TILE = 640
_COPY_ROWS = 128
_CROWS = _NF_PAD // 128


def _sc_segment_sum(x, edge_src, edge_dst):
  e = edge_src.shape[0]
  d = x.shape[1]
  nw = _NC * _NS
  per_tile = e // nw
  n_chunks = per_tile // _CHUNK

  mesh = plsc.VectorSubcoreMesh(core_axis_name="c", subcore_axis_name="s")

  @functools.partial(
      pl.kernel,
      mesh=mesh,
      compiler_params=pltpu.CompilerParams(needs_layout_passes=False),
      out_type=[
          jax.ShapeDtypeStruct((_NC, _NF_PAD, d), jnp.float32),
          jax.ShapeDtypeStruct((_NC, _CROWS, 128), jnp.float32),
      ],
      scratch_types=[
          pltpu.VMEM_SHARED((_NF_PAD, d), jnp.float32),
          pltpu.VMEM_SHARED((_CROWS, 128), jnp.float32),
          pltpu.VMEM((3, _CHUNK), jnp.int32),
          pltpu.VMEM((6, _CHUNK), jnp.int32),
          pltpu.VMEM((3, _CHUNK, d), jnp.float32),
          pltpu.VMEM((_CROWS + 1, 128), jnp.float32),
          pltpu.VMEM((_CROWS,), jnp.int32),
      ] + [pltpu.SemaphoreType.DMA] * 15,
  )
  def k(x_hbm, src_hbm, dst_hbm, out_hbm, cnt_hbm, acc_sh, cnt_sh, sidx_v,
        didx_v, rows2_v, hist_v, rowid_v, *sems):
    cid = lax.axis_index("c")
    sid = lax.axis_index("s")
    wid = cid * _NS + sid
    row0 = sid * _ROWS_PER_TILE
    isem = list(sems[0:3])
    dsem = list(sems[3:9])
    gsem = list(sems[9:12])
    ssem = list(sems[12:15])

    iota = lax.iota(jnp.int32, _LANES)
    zvec = jnp.zeros((_LANES,), jnp.float32)
    lane15 = iota == _LANES - 1
    shift_ix = [jnp.maximum(iota - k, 0)[:, None] for k in (1, 2, 4, 8)]
    nxt_ix = jnp.minimum(iota + 1, _LANES - 1)[:, None]
    ones16 = jnp.ones((_LANES,), jnp.int32)
    zeros16 = jnp.zeros((_LANES,), jnp.int32)
    gd = lax.GatherDimensionNumbers(
        offset_dims=(), collapsed_slice_dims=(0,), start_index_map=(0,))

    def shift(v, ix):
      return lax.gather(v, ix, gd, (1,),
                        mode=lax.GatherScatterMode.PROMISE_IN_BOUNDS)

    def zero_row(r, _):
      for j in range(d // _LANES):
        rows2_v[0, r, pl.ds(j * _LANES, _LANES)] = zvec
      return 0

    lax.fori_loop(0, _CHUNK, zero_row, 0)
    for blk in range(_ROWS_PER_TILE // _CHUNK):
      pltpu.sync_copy(rows2_v.at[0],
                      acc_sh.at[pl.ds(row0 + blk * _CHUNK, _CHUNK), :])

    def zero_hist(r, _):
      for j in range(128 // _LANES):
        hist_v[r, pl.ds(j * _LANES, _LANES)] = zvec
      return 0

    lax.fori_loop(0, _CROWS + 1, zero_hist, 0)
    for g in range(_CROWS // _LANES):
      rowid_v[pl.ds(g * _LANES, _LANES)] = iota + g * _LANES

    @pl.when(sid == 0)
    def _():
      pltpu.sync_copy(rows2_v.at[0, pl.ds(0, _CROWS), :], cnt_sh)

    plsc.subcore_barrier()

    def counts(slot):
      for g in range(_CHUNK // _LANES):
        dstv = didx_v[slot, pl.ds(g * _LANES, _LANES)]
        prv = shift(dstv, shift_ix[0])
        nxt = shift(dstv, nxt_ix)
        is_last = (dstv != nxt) | lane15
        m = jnp.where((dstv == prv) & (iota > 0), ones16, zeros16)
        c = ones16
        for ix in shift_ix:
          c = c + m * shift(c, ix)
          m = m * shift(m, ix)
        cnt = c.astype(jnp.float32)
        rows = jnp.where(is_last, lax.shift_right_logical(dstv, 7),
                         jnp.full((_LANES,), _CROWS, jnp.int32))
        plsc.addupdate_scatter(hist_v, [rows, lax.bitwise_and(dstv, 127)],
                               cnt)

    base = wid * n_chunks

    def fire_sidx(c, b):
      pltpu.async_copy(src_hbm.at[base + c], sidx_v.at[b], isem[b])

    def wait_sidx(c, b):
      pltpu.make_async_copy(src_hbm.at[base + c], sidx_v.at[b],
                            isem[b]).wait()

    def fire_didx(c, slot):
      pltpu.async_copy(dst_hbm.at[base + c], didx_v.at[slot], dsem[slot])

    def wait_didx(c, slot):
      pltpu.make_async_copy(dst_hbm.at[base + c], didx_v.at[slot],
                            dsem[slot]).wait()

    def fire_gather(b):
      pltpu.async_copy(x_hbm.at[sidx_v.at[b]], rows2_v.at[b], gsem[b])

    def wait_gather(b):
      pltpu.make_async_copy(x_hbm.at[sidx_v.at[b]], rows2_v.at[b],
                            gsem[b]).wait()

    def fire_scatter(slot, b):
      pltpu.async_copy(rows2_v.at[b], acc_sh.at[didx_v.at[slot]], ssem[b],
                       add=True)

    def wait_scatter(slot, b):
      pltpu.make_async_copy(rows2_v.at[b], acc_sh.at[didx_v.at[slot]],
                            ssem[b]).wait()

    def step(c, j, dyn):
      b = j % 3
      slot = j % 6
      wait_gather(b)

      def do_sidx():
        fire_sidx(c + 3, b)

      def do_didx():
        fire_didx(c + 6, slot)

      def do_gather():
        wait_sidx(c + 3, b)
        fire_gather(b)

      if dyn:
        pl.when(c + 3 < n_chunks)(do_sidx)
      elif c + 3 < n_chunks:
        do_sidx()
      wait_didx(c, slot)
      fire_scatter(slot, b)
      counts(slot)
      wait_scatter(slot, b)
      if dyn:
        pl.when(c + 6 < n_chunks)(do_didx)
      elif c + 6 < n_chunks:
        do_didx()
      if dyn:
        pl.when(c + 3 < n_chunks)(do_gather)
      elif c + 3 < n_chunks:
        do_gather()

    for j in range(3):
      fire_sidx(j, j)
    for j in range(6):
      fire_didx(j, j)
    for j in range(3):
      wait_sidx(j, j)
      fire_gather(j)

    def hex_body(k6, _):
      c0 = k6 * 6
      for j in range(6):
        step(c0 + j, j, True)
      return 0

    n_loop = n_chunks // 6
    lax.fori_loop(0, n_loop, hex_body, 0)
    for c in range(n_loop * 6, n_chunks):
      step(c, c % 6, False)

    pltpu.sync_copy(hist_v.at[pl.ds(0, _CROWS), :], cnt_sh.at[rowid_v],
                    add=True)
    plsc.subcore_barrier()

    pltpu.sync_copy(acc_sh.at[pl.ds(row0, _ROWS_PER_TILE), :],
                    out_hbm.at[cid, pl.ds(row0, _ROWS_PER_TILE), :])

    @pl.when(sid == 0)
    def _():
      pltpu.sync_copy(cnt_sh, cnt_hbm.at[cid])

  src2 = edge_src.reshape(nw * n_chunks, _CHUNK)
  dst2 = edge_dst.reshape(nw * n_chunks, _CHUNK)
  return k(x, src2, dst2)


def _tc_head(partials, counts, x_skip, w1, w2, b2):
  n_fine, d = x_skip.shape
  bm = 2000
  grid = (n_fine // bm,)

  def body(s_ref, c_ref, xs_ref, w1_ref, w2_ref, b_ref, o_ref):
    s = s_ref[0] + s_ref[1]
    cnt = c_ref[0] + c_ref[1]
    agg = s * (1.0 / jnp.maximum(cnt, 1.0))
    acc = jnp.dot(agg, w1_ref[...], preferred_element_type=jnp.float32)
    acc = acc + jnp.dot(xs_ref[...], w2_ref[...],
                        preferred_element_type=jnp.float32)
    o_ref[...] = jnp.maximum(acc + b_ref[...], 0.0)

  return pl.pallas_call(
      body,
      grid=grid,
      in_specs=[
          pl.BlockSpec((_NC, bm, d), lambda i: (0, i, 0)),
          pl.BlockSpec((_NC, bm, 1), lambda i: (0, i, 0)),
          pl.BlockSpec((bm, d), lambda i: (i, 0)),
          pl.BlockSpec((d, d), lambda i: (0, 0)),
          pl.BlockSpec((d, d), lambda i: (0, 0)),
          pl.BlockSpec((1, d), lambda i: (0, 0)),
      ],
      out_specs=pl.BlockSpec((bm, d), lambda i: (i, 0)),
      out_shape=jax.ShapeDtypeStruct((n_fine, d), jnp.float32),
  )(partials, counts, x_skip, w1, w2, b2)


def kernel(x, pos, x_skip, pos_skip, W, b, batch, batch_skip, edge_src,
           edge_dst):
  n_fine, d = x_skip.shape
  partials, cnt_packed = _sc_segment_sum(x, edge_src, edge_dst)
  counts = cnt_packed.reshape(_NC, _NF_PAD, 1)
  return _tc_head(partials, counts, x_skip, W[:d], W[d:], b.reshape(1, -1))
